# baseline (speedup 1.0000x reference)
import functools

import jax
import jax.numpy as jnp
from jax import lax
from jax.experimental import pallas as pl
from jax.experimental.pallas import tpu as pltpu
from jax.experimental.pallas import tpu_sc as plsc

F = 64
B = 16384
SEG = 128


def _gmf_sc(x1r, x2r, user_table, item_table, wb):
    info = plsc.get_sparse_core_info()
    nw = info.num_cores * info.num_subcores
    b_per_w = B // nw
    n_seg = b_per_w // SEG

    mesh = plsc.VectorSubcoreMesh(core_axis_name="c", subcore_axis_name="s")

    @functools.partial(
        pl.kernel,
        mesh=mesh,
        out_type=jax.ShapeDtypeStruct((B,), jnp.float32),
        compiler_params=pltpu.CompilerParams(use_tc_tiling_on_sc=False),
        scratch_types=[
            pltpu.VMEM((n_seg, SEG), jnp.int32),
            pltpu.VMEM((n_seg, SEG), jnp.int32),
            pltpu.VMEM((b_per_w, F), jnp.float32),
            pltpu.VMEM((b_per_w, F), jnp.float32),
            pltpu.VMEM((b_per_w,), jnp.float32),
            pltpu.VMEM((F + 16,), jnp.float32),
            pltpu.SemaphoreType.DMA,
        ],
    )
    def k(x1_hbm, x2_hbm, u_hbm, v_hbm, wb_hbm, out_hbm,
          idx1_v, idx2_v, u_v, v_v, out_v, wb_v, sem):
        wid = lax.axis_index("s") * info.num_cores + lax.axis_index("c")
        base = wid * b_per_w

        pltpu.sync_copy(x1_hbm.at[pl.ds(wid * n_seg, n_seg)], idx1_v)
        pltpu.sync_copy(x2_hbm.at[pl.ds(wid * n_seg, n_seg)], idx2_v)

        copies = []
        for j in range(n_seg):
            copies.append(pltpu.async_copy(
                u_hbm.at[idx1_v.at[j]], u_v.at[pl.ds(j * SEG, SEG)], sem))
            copies.append(pltpu.async_copy(
                v_hbm.at[idx2_v.at[j]], v_v.at[pl.ds(j * SEG, SEG)], sem))
        pltpu.sync_copy(wb_hbm, wb_v)
        for c in copies:
            c.wait()

        w0 = wb_v[pl.ds(0, 16)]
        w1 = wb_v[pl.ds(16, 16)]
        w2 = wb_v[pl.ds(32, 16)]
        w3 = wb_v[pl.ds(48, 16)]

        bias = wb_v[pl.ds(F, 16)]
        lane = lax.iota(jnp.int32, 16)
        perms = [jnp.bitwise_xor(lane, sh) for sh in (8, 4, 2, 1)]

        def lanesum(v):
            for p in perms:
                v = v + v.at[p].get(mode="promise_in_bounds")
            return v

        def group_body(g, carry):
            base_r = pl.multiple_of(g * 16, 16)
            acc = jnp.zeros((16,), jnp.float32)
            for r in range(16):
                i = base_r + r
                s = (u_v[i, pl.ds(0, 16)] * v_v[i, pl.ds(0, 16)]) * w0
                s = s + (u_v[i, pl.ds(16, 16)] * v_v[i, pl.ds(16, 16)]) * w1
                s = s + (u_v[i, pl.ds(32, 16)] * v_v[i, pl.ds(32, 16)]) * w2
                s = s + (u_v[i, pl.ds(48, 16)] * v_v[i, pl.ds(48, 16)]) * w3
                acc = jnp.where(lane == r, lanesum(s), acc)
            x = acc + bias
            out_v[pl.ds(base_r, 16)] = 1.0 / (1.0 + jnp.exp(-x))
            return carry

        lax.fori_loop(0, b_per_w // 16, group_body, 0)

        pltpu.sync_copy(out_v, out_hbm.at[pl.ds(base, b_per_w)])

    return k(x1r, x2r, user_table, item_table, wb)


def kernel(x1, x2, user_table, item_table, W, b):
    x1r = x1.reshape(B // SEG, SEG)
    x2r = x2.reshape(B // SEG, SEG)
    wb = jnp.concatenate([W.reshape(F), jnp.broadcast_to(b, (16,))])
    out = _gmf_sc(x1r, x2r, user_table, item_table, wb)
    return out.reshape(B, 1)
